# Initial kernel scaffold; baseline (speedup 1.0000x reference)
import jax
import jax.numpy as jnp
from jax.experimental import pallas as pl


def kernel(input_h, input_e, edge_index, W_f, We_f, a_f, W_r, We_r, a_r, gamma_h, beta_h, gamma_e, beta_e):
    raise NotImplementedError("write your pallas kernel here")


# speedup vs baseline: 2.9759x; 2.9759x over previous
import functools

import jax
import jax.numpy as jnp
from jax import lax
from jax.experimental import pallas as pl
from jax.experimental.pallas import tpu as pltpu
from jax.experimental.pallas import tpu_sc as plsc

ALPHA = 0.2
EPS = 1e-5
NC, NS, L = 2, 16, 16
NW = NC * NS


def _edge_mm_body(e_ref, wf_ref, wr_ref, af_ref, ar_ref,
                  ef_ref, er_ref, tf_ref, tr_ref):
    e = e_ref[...]
    ef = jnp.dot(e, wf_ref[...], preferred_element_type=jnp.float32)
    er = jnp.dot(e, wr_ref[...], preferred_element_type=jnp.float32)
    ef_ref[...] = ef
    er_ref[...] = er
    tf_ref[...] = jnp.dot(ef, af_ref[...], preferred_element_type=jnp.float32)
    tr_ref[...] = jnp.dot(er, ar_ref[...], preferred_element_type=jnp.float32)


def _edge_mm(input_e, We_f, We_r, af3, ar3, be=1280):
    E, D = input_e.shape
    wspec = pl.BlockSpec((D, D), lambda i: (0, 0))
    aspec = pl.BlockSpec((D, 1), lambda i: (0, 0))
    bspec = pl.BlockSpec((be, D), lambda i: (i, 0))
    tspec = pl.BlockSpec((be, 1), lambda i: (i, 0))
    return pl.pallas_call(
        _edge_mm_body,
        grid=(E // be,),
        in_specs=[bspec, wspec, wspec, aspec, aspec],
        out_specs=[bspec, bspec, tspec, tspec],
        out_shape=[jax.ShapeDtypeStruct((E, D), jnp.float32),
                   jax.ShapeDtypeStruct((E, D), jnp.float32),
                   jax.ShapeDtypeStruct((E, 1), jnp.float32),
                   jax.ShapeDtypeStruct((E, 1), jnp.float32)],
    )(input_e, We_f, We_r, af3, ar3)


def _node_mm_body(h_ref, wf_ref, wr_ref, af1_ref, af2_ref, ar1_ref, ar2_ref,
                  hf_ref, hr_ref, s1f_ref, s2f_ref, s1r_ref, s2r_ref):
    h = h_ref[...]
    hf = jnp.dot(h, wf_ref[...], preferred_element_type=jnp.float32)
    hr = jnp.dot(h, wr_ref[...], preferred_element_type=jnp.float32)
    hf_ref[...] = hf
    hr_ref[...] = hr
    s1f_ref[...] = jnp.dot(hf, af1_ref[...], preferred_element_type=jnp.float32)
    s2f_ref[...] = jnp.dot(hf, af2_ref[...], preferred_element_type=jnp.float32)
    s1r_ref[...] = jnp.dot(hr, ar1_ref[...], preferred_element_type=jnp.float32)
    s2r_ref[...] = jnp.dot(hr, ar2_ref[...], preferred_element_type=jnp.float32)


def _node_mm(input_h, W_f, W_r, af1, af2, ar1, ar2):
    N, D = input_h.shape
    nd = jax.ShapeDtypeStruct((N, D), jnp.float32)
    n1 = jax.ShapeDtypeStruct((N, 1), jnp.float32)
    return pl.pallas_call(
        _node_mm_body,
        out_shape=[nd, nd, n1, n1, n1, n1],
    )(input_h, W_f, W_r, af1, af2, ar1, ar2)


def _make_sc_att(E, N):
    EPW = E // NW
    NG = EPW // L

    mesh = plsc.VectorSubcoreMesh(core_axis_name="c", subcore_axis_name="s",
                                  num_cores=NC, num_subcores=NS)

    def body(srcm_h, dstm_h, tf_h, tr_h, s1f_h, s2f_h, s1r_h, s2r_h,
             exf_o, exr_o,
             src_v, dst_v, t_v, ex_v, s1_v, s2_v):
        cid = lax.axis_index("c")
        sid = lax.axis_index("s")
        wid = cid * NS + sid

        pltpu.sync_copy(srcm_h.at[wid], src_v)
        pltpu.sync_copy(dstm_h.at[wid], dst_v)

        def run_direction(t_h, sA_h, sB_h, ex_o):
            pltpu.sync_copy(t_h.at[wid], t_v)
            pltpu.sync_copy(sA_h, s1_v)
            pltpu.sync_copy(sB_h, s2_v)

            def grp(g, carry):
                si = src_v[pl.ds(g * L, L)]
                di = dst_v[pl.ds(g * L, L)]
                a = (plsc.load_gather(s1_v, [si])
                     + plsc.load_gather(s2_v, [di])
                     + t_v[pl.ds(g * L, L)])
                a = jnp.where(a > 0.0, a, a * ALPHA)
                ex_v[pl.ds(g * L, L)] = jnp.exp(a)
                return carry

            lax.fori_loop(0, NG, grp, 0)
            pltpu.sync_copy(ex_v, ex_o.at[wid])

        run_direction(tf_h, s1f_h, s2f_h, exf_o)
        run_direction(tr_h, s2r_h, s1r_h, exr_o)

    ew = jax.ShapeDtypeStruct((NW, EPW), jnp.float32)
    return pl.kernel(
        body,
        out_type=[ew, ew],
        mesh=mesh,
        compiler_params=pltpu.CompilerParams(needs_layout_passes=False),
        scratch_types=[
            pltpu.VMEM((EPW,), jnp.int32),
            pltpu.VMEM((EPW,), jnp.int32),
            pltpu.VMEM((EPW,), jnp.float32),
            pltpu.VMEM((EPW,), jnp.float32),
            pltpu.VMEM((N,), jnp.float32),
            pltpu.VMEM((N,), jnp.float32),
        ],
    )


def _scale_body(ef_ref, er_ref, exf_ref, exr_ref, pf_ref, pr_ref):
    pf_ref[...] = ef_ref[...] * exf_ref[...]
    pr_ref[...] = er_ref[...] * exr_ref[...]


def _scale(e_f, e_r, exf, exr, be=2560):
    E, D = e_f.shape
    bspec = pl.BlockSpec((be, D), lambda i: (i, 0))
    vspec = pl.BlockSpec((be, 1), lambda i: (i, 0))
    ed = jax.ShapeDtypeStruct((E, D), jnp.float32)
    return pl.pallas_call(
        _scale_body,
        grid=(E // be,),
        in_specs=[bspec, bspec, vspec, vspec],
        out_specs=[bspec, bspec],
        out_shape=[ed, ed],
    )(e_f, e_r, exf.reshape(E, 1), exr.reshape(E, 1))


def _make_sc_scatter(E, N, D):
    EPW = E // NW
    C = 80
    NCH = EPW // C
    NPW = -(-(N // NS) // 8) * 8
    NP = NPW * NS
    ZR = 8
    ZSTEPS = NPW // ZR
    KD = D // L
    P = 8

    mesh = plsc.VectorSubcoreMesh(core_axis_name="c", subcore_axis_name="s",
                                  num_cores=NC, num_subcores=NS)

    def body(srcm_h, dstm_h, exf_h, exr_h, pf_h, pr_h,
             hsf_o, sf_o, hsr_o, sr_o,
             idx_c, ex_c, row_v, pad_v, zrow_v, zpad_v,
             acc_h, acc_s):
        cid = lax.axis_index("c")
        sid = lax.axis_index("s")
        wid = cid * NS + sid
        ebase = wid * EPW
        r0 = sid * NPW

        zf = jnp.zeros((L,), jnp.float32)
        for r in range(ZR):
            for k in range(KD):
                zrow_v[r, pl.ds(k * L, L)] = zf

        iota = lax.iota(jnp.int32, L)
        zidx = jnp.zeros((L,), jnp.int32)
        prow = jnp.right_shift(iota, 3)
        pcol = jnp.bitwise_and(iota, 7)
        for b in range(ZR // 2):
            plsc.store_scatter(zpad_v, [prow + 2 * b, pcol], zf)
        for b in range(C // 2):
            plsc.store_scatter(pad_v, [prow + 2 * b, pcol], zf)

        def zero_stripe():
            def zq(q, carry):
                pltpu.sync_copy(zrow_v, acc_h.at[pl.ds(r0 + q * ZR, ZR)])
                pltpu.sync_copy(zpad_v, acc_s.at[pl.ds(r0 + q * ZR, ZR)])
                return carry
            lax.fori_loop(0, ZSTEPS, zq, 0)

        def run_direction(idxm_h, ex_h, p_h, hs_o, s_o):
            zero_stripe()
            plsc.subcore_barrier()

            def chunk(j, carry):
                pltpu.sync_copy(idxm_h.at[wid, j], idx_c)
                pltpu.sync_copy(ex_h.at[wid, j], ex_c)
                pltpu.sync_copy(p_h.at[pl.ds(ebase + j * C, C)], row_v)
                for g in range(C // L):
                    plsc.store_scatter(pad_v, [iota + g * L, zidx],
                                       ex_c[pl.ds(g * L, L)])
                pltpu.sync_copy(row_v, acc_h.at[idx_c], add=True)
                pltpu.sync_copy(pad_v, acc_s.at[idx_c], add=True)
                return carry

            lax.fori_loop(0, NCH, chunk, 0)
            plsc.subcore_barrier()

            pltpu.sync_copy(acc_h.at[pl.ds(r0, NPW)],
                            hs_o.at[cid, pl.ds(r0, NPW)])
            pltpu.sync_copy(acc_s.at[pl.ds(r0, NPW)],
                            s_o.at[cid, pl.ds(r0, NPW)])

        run_direction(dstm_h, exf_h, pf_h, hsf_o, sf_o)
        run_direction(srcm_h, exr_h, pr_h, hsr_o, sr_o)

    nd = jax.ShapeDtypeStruct((NC, NP, D), jnp.float32)
    n16 = jax.ShapeDtypeStruct((NC, NP, P), jnp.float32)
    return pl.kernel(
        body,
        out_type=[nd, n16, nd, n16],
        mesh=mesh,
        compiler_params=pltpu.CompilerParams(needs_layout_passes=False),
        scratch_types=[
            pltpu.VMEM((C,), jnp.int32),
            pltpu.VMEM((C,), jnp.float32),
            pltpu.VMEM((C, D), jnp.float32),
            pltpu.VMEM((C, P), jnp.float32),
            pltpu.VMEM((ZR, D), jnp.float32),
            pltpu.VMEM((ZR, P), jnp.float32),
            pltpu.VMEM_SHARED((NP, D), jnp.float32),
            pltpu.VMEM_SHARED((NP, P), jnp.float32),
        ],
    )


def _combine_body(hsf_ref, sf_ref, hsr_ref, sr_ref, hf_ref,
                  oh_ref, st_ref):
    s_f = jnp.sum(sf_ref[0] + sf_ref[1], axis=1, keepdims=True)
    s_r = jnp.sum(sr_ref[0] + sr_ref[1], axis=1, keepdims=True)
    hs_f = (hsf_ref[0] + hsf_ref[1]) / jnp.where(s_f > 0.0, s_f, 1.0)
    hs_r = (hsr_ref[0] + hsr_ref[1]) / jnp.where(s_r > 0.0, s_r, 1.0)
    oh = hs_f + hs_r + hf_ref[...]
    oh_ref[...] = oh
    st_ref[0, 0, :] = jnp.sum(oh, axis=0)
    st_ref[0, 1, :] = jnp.sum(oh * oh, axis=0)


def _combine(hsf, sfp, hsr, srp, h_f, bn=2000):
    NC_, NP, D = hsf.shape
    P = sfp.shape[2]
    N = h_f.shape[0]
    nb = N // bn
    pspec = pl.BlockSpec((NC_, bn, D), lambda i: (0, i, 0))
    sspec = pl.BlockSpec((NC_, bn, P), lambda i: (0, i, 0))
    bspec = pl.BlockSpec((bn, D), lambda i: (i, 0))
    stspec = pl.BlockSpec((1, 2, D), lambda i: (i, 0, 0))
    return pl.pallas_call(
        _combine_body,
        grid=(nb,),
        in_specs=[pspec, sspec, pspec, sspec, bspec],
        out_specs=[bspec, stspec],
        out_shape=[jax.ShapeDtypeStruct((N, D), jnp.float32),
                   jax.ShapeDtypeStruct((nb, 2, D), jnp.float32)],
    )(hsf, sfp, hsr, srp, h_f)


def _make_sc_z(E, N, D):
    EPW = E // NW
    C = 40
    NCH = EPW // C
    KD = D // L

    mesh = plsc.VectorSubcoreMesh(core_axis_name="c", subcore_axis_name="s",
                                  num_cores=NC, num_subcores=NS)

    def body(oh_h, srcm_h, dstm_h, ef_h, z_o, st_o,
             src_v, dst_v, buf_a, buf_b, buf_c, stat_v):
        cid = lax.axis_index("c")
        sid = lax.axis_index("s")
        wid = cid * NS + sid
        ebase = wid * EPW

        pltpu.sync_copy(srcm_h.at[wid], src_v)
        pltpu.sync_copy(dstm_h.at[wid], dst_v)

        init = tuple(jnp.zeros((L,), jnp.float32) for _ in range(2 * KD))

        def ph(j, carry):
            pltpu.sync_copy(oh_h.at[src_v.at[j]], buf_a)
            pltpu.sync_copy(oh_h.at[dst_v.at[j]], buf_b)
            pltpu.sync_copy(ef_h.at[pl.ds(ebase + j * C, C)], buf_c)
            acc = list(carry)
            for r in range(C):
                for k in range(KD):
                    z = (buf_a[r, pl.ds(k * L, L)] + buf_b[r, pl.ds(k * L, L)]
                         + buf_c[r, pl.ds(k * L, L)])
                    buf_c[r, pl.ds(k * L, L)] = z
                    acc[k] = acc[k] + z
                    acc[KD + k] = acc[KD + k] + z * z
            pltpu.sync_copy(buf_c, z_o.at[pl.ds(ebase + j * C, C)])
            return tuple(acc)

        acc = lax.fori_loop(0, NCH, ph, init)
        for k in range(KD):
            stat_v[0, pl.ds(k * L, L)] = acc[k]
            stat_v[1, pl.ds(k * L, L)] = acc[KD + k]
        pltpu.sync_copy(stat_v, st_o.at[wid])

    return pl.kernel(
        body,
        out_type=[jax.ShapeDtypeStruct((E, D), jnp.float32),
                  jax.ShapeDtypeStruct((NW, 2, D), jnp.float32)],
        mesh=mesh,
        scratch_types=[
            pltpu.VMEM((NCH, C), jnp.int32),
            pltpu.VMEM((NCH, C), jnp.int32),
            pltpu.VMEM((C, D), jnp.float32),
            pltpu.VMEM((C, D), jnp.float32),
            pltpu.VMEM((C, D), jnp.float32),
            pltpu.VMEM((2, D), jnp.float32),
        ],
    )


def _bn_res_body(count, z_ref, res_ref, st_ref, g_ref, b_ref, out_ref):
    st = jnp.sum(st_ref[...], axis=0)
    mu = st[0:1, :] * (1.0 / count)
    msq = st[1:2, :] * (1.0 / count)
    var = msq - mu * mu
    y = (z_ref[...] - mu) * lax.rsqrt(var + EPS) * g_ref[...] + b_ref[...]
    out_ref[...] = jnp.maximum(y, 0.0) + res_ref[...]


def _bn_res(z, resid, stats, gamma, beta, blk):
    M, D = z.shape
    ns = stats.shape[0]
    bspec = pl.BlockSpec((blk, D), lambda i: (i, 0))
    sspec = pl.BlockSpec((ns, 2, D), lambda i: (0, 0, 0))
    gspec = pl.BlockSpec((1, D), lambda i: (0, 0))
    return pl.pallas_call(
        functools.partial(_bn_res_body, float(M)),
        grid=(M // blk,),
        in_specs=[bspec, bspec, sspec, gspec, gspec],
        out_specs=bspec,
        out_shape=jax.ShapeDtypeStruct((M, D), jnp.float32),
    )(z, resid, stats, gamma.reshape(1, D), beta.reshape(1, D))


def kernel(input_h, input_e, edge_index, W_f, We_f, a_f, W_r, We_r, a_r,
           gamma_h, beta_h, gamma_e, beta_e):
    N, D = input_h.shape
    E = input_e.shape[0]
    assert E % NW == 0 and N % NS == 0 and D % L == 0

    src = edge_index[0]
    dst = edge_index[1]

    e_f, e_r, t_f, t_r = _edge_mm(input_e, We_f, We_r,
                                  a_f[2 * D:], a_r[2 * D:])
    h_f, h_r, s1f, s2f, s1r, s2r = _node_mm(
        input_h, W_f, W_r, a_f[:D], a_f[D:2 * D], a_r[:D], a_r[D:2 * D])

    epw = E // NW
    sc_att = _make_sc_att(E, N)
    exf, exr = sc_att(
        src.reshape(NW, epw), dst.reshape(NW, epw),
        t_f.reshape(NW, epw), t_r.reshape(NW, epw),
        s1f.reshape(N), s2f.reshape(N), s1r.reshape(N), s2r.reshape(N))

    NPW = -(-(N // NS) // 8) * 8
    NP = NPW * NS
    P = 8
    exf_l = exf.reshape(E)
    exr_l = exr.reshape(E)
    zpart = jnp.zeros((NP, D), jnp.float32)
    hs_f = jax.ops.segment_sum(exf_l[:, None] * e_f, dst, num_segments=NP)
    hs_r = jax.ops.segment_sum(exr_l[:, None] * e_r, src, num_segments=NP)
    s_f = jax.ops.segment_sum(exf_l, dst, num_segments=NP)
    s_r = jax.ops.segment_sum(exr_l, src, num_segments=NP)
    zp = jnp.zeros((NP, P), jnp.float32)
    sfp = jnp.stack([zp.at[:, 0].set(s_f), zp])
    srp = jnp.stack([zp.at[:, 0].set(s_r), zp])
    hsf = jnp.stack([hs_f, zpart])
    hsr = jnp.stack([hs_r, zpart])

    output_h, h_stats = _combine(hsf, sfp, hsr, srp, h_f)
    out_h = _bn_res(output_h, input_h, h_stats, gamma_h, beta_h, blk=2000)

    c2 = 40
    sc_z = _make_sc_z(E, N, D)
    z, stats = sc_z(output_h, src.reshape(NW, epw // c2, c2),
                    dst.reshape(NW, epw // c2, c2), e_f)

    e_ji = _bn_res(z, input_e, stats, gamma_e, beta_e, blk=1280)
    return (out_h, e_ji)
